# baseline (speedup 1.0000x reference)
import functools

import jax
import jax.numpy as jnp
from jax import lax
from jax.experimental import pallas as pl
from jax.experimental.pallas import tpu as pltpu
from jax.experimental.pallas import tpu_sc as plsc

_N, _E, _D, _G = 10000, 320000, 128, 64
_NC, _NS = 2, 16
_CH = 125
_ROWS = _E // _CH
_RPT = _ROWS // (_NC * _NS)
_HR = _RPT // 2
_ZC = 40
_NZC = _N // _ZC
_ACH = 25
_ARND = 40
_ANR = 10
_NBUF = 8
_B = 1000
_NSTEPS = _N // _B

_mesh = plsc.VectorSubcoreMesh(
    core_axis_name="c", subcore_axis_name="s", num_cores=_NC, num_subcores=_NS)


def _agg_body(h_hbm, src_hbm, dst_hbm, zr_hbm, out_hbm,
              sidx, didx, *rest):
    rows = rest[:_NBUF]
    acc = rest[_NBUF]
    gsems = rest[_NBUF + 1:2 * _NBUF + 1]
    ssems = rest[2 * _NBUF + 1:]
    rows0 = rows[0]
    cid = lax.axis_index("c")
    sid = lax.axis_index("s")
    wid = cid * _NS + sid
    pltpu.sync_copy(zr_hbm, rows0.at[pl.ds(0, _ZC)])
    pltpu.sync_copy(src_hbm.at[wid, 0], sidx)
    pltpu.sync_copy(dst_hbm.at[wid, 0], didx)
    for b in range(1, _NBUF - 1):
        pltpu.async_copy(h_hbm.at[sidx.at[b]], rows[b], gsems[b])

    @pl.loop(sid, _NZC, step=_NS)
    def _zero(c):
        pltpu.sync_copy(rows0.at[pl.ds(0, _ZC)], acc.at[pl.ds(c * _ZC, _ZC)])

    pltpu.async_copy(h_hbm.at[sidx.at[0]], rows0, gsems[0])
    plsc.subcore_barrier()

    for r in range(_ANR):
        if r > 0:
            pltpu.sync_copy(src_hbm.at[wid, r], sidx)
            pltpu.sync_copy(dst_hbm.at[wid, r], didx)
            for b in range(_NBUF - 1):
                pltpu.async_copy(h_hbm.at[sidx.at[b]], rows[b], gsems[b])

        @pl.loop(0, _ARND, step=_NBUF)
        def _main(j):
            for b in range(_NBUF):
                c = j + b
                pltpu.make_async_copy(h_hbm.at[sidx.at[c]], rows[b],
                                      gsems[b]).wait()
                nb = (b + _NBUF - 1) % _NBUF

                @pl.when(c + _NBUF - 1 < _ARND)
                def _():
                    if b == 0:
                        @pl.when(j > 0)
                        def _():
                            pltpu.make_async_copy(
                                rows[nb], acc.at[didx.at[c]],
                                ssems[nb]).wait()
                    else:
                        pltpu.make_async_copy(
                            rows[nb], acc.at[didx.at[c]], ssems[nb]).wait()
                    pltpu.async_copy(h_hbm.at[sidx.at[c + _NBUF - 1]],
                                     rows[nb], gsems[nb])

                pltpu.async_copy(rows[b], acc.at[didx.at[c]], ssems[b],
                                 add=True)

        for b in range(_NBUF):
            pltpu.make_async_copy(rows[b], acc.at[didx.at[0]],
                                  ssems[b]).wait()

    plsc.subcore_barrier()

    @pl.loop(sid, _NZC, step=_NS)
    def _wb(c):
        pltpu.sync_copy(acc.at[pl.ds(c * _ZC, _ZC)],
                        out_hbm.at[cid, pl.ds(c * _ZC, _ZC)])


_sc_agg = functools.partial(
    pl.kernel, _agg_body,
    out_type=jax.ShapeDtypeStruct((_NC, _N, _D), jnp.float32),
    mesh=_mesh,
    scratch_types=(
        [pltpu.VMEM((_ARND, _ACH), jnp.int32)] * 2
        + [pltpu.VMEM((_ACH, _D), jnp.float32)] * _NBUF
        + [pltpu.VMEM_SHARED((_N, _D), jnp.float32)]
        + [pltpu.SemaphoreType.DMA] * (2 * _NBUF)
    ),
)()


def _deg_body(dst_hbm, zr_hbm, on_hbm, out_hbm, didx, rows, acc):
    cid = lax.axis_index("c")
    sid = lax.axis_index("s")
    pltpu.sync_copy(zr_hbm, rows.at[pl.ds(0, _ZC)])

    @pl.loop(sid, _NZC, step=_NS)
    def _zero(c):
        pltpu.sync_copy(rows.at[pl.ds(0, _ZC)], acc.at[pl.ds(c * _ZC, _ZC)])

    pltpu.sync_copy(on_hbm, rows)
    plsc.subcore_barrier()

    for half in range(2):
        pltpu.sync_copy(dst_hbm.at[cid * _NS + sid, pl.ds(half * _HR, _HR)],
                        didx)

        @pl.loop(0, _HR)
        def _main(j):
            pltpu.sync_copy(rows, acc.at[didx.at[j]], add=True)

    plsc.subcore_barrier()

    @pl.loop(sid, _NZC, step=_NS)
    def _wb(c):
        pltpu.sync_copy(acc.at[pl.ds(c * _ZC, _ZC)],
                        out_hbm.at[cid, pl.ds(c * _ZC, _ZC)])


_sc_deg = functools.partial(
    pl.kernel, _deg_body,
    out_type=jax.ShapeDtypeStruct((_NC, _N, _D), jnp.float32),
    mesh=_mesh,
    scratch_types=[
        pltpu.VMEM((_HR, _CH), jnp.int32),
        pltpu.VMEM((_CH, _D), jnp.float32),
        pltpu.VMEM_SHARED((_N, _D), jnp.float32),
    ],
)()


def _dense_body(relu, p_ref, d_ref, h_ref, wl_ref, bl_ref, wr_ref, o_ref):
    agg = p_ref[0] + p_ref[1]
    deg = d_ref[0, :, 0] + d_ref[1, :, 0]
    mean = agg * (1.0 / jnp.maximum(deg, 1.0))[:, None]
    out = (lax.dot_general(mean, wl_ref[...], (((1,), (1,)), ((), ())),
                           preferred_element_type=jnp.float32)
           + lax.dot_general(h_ref[...], wr_ref[...], (((1,), (1,)), ((), ())),
                             preferred_element_type=jnp.float32)
           + bl_ref[...])
    if relu:
        out = jnp.maximum(out, 0.0)
    o_ref[...] = out


def _tc_layer(p, degt, h, wl, bl, wr, relu):
    return pl.pallas_call(
        functools.partial(_dense_body, relu),
        grid=(_NSTEPS,),
        in_specs=[
            pl.BlockSpec((_NC, _B, _D), lambda i: (0, i, 0)),
            pl.BlockSpec((_NC, _B, _D), lambda i: (0, i, 0)),
            pl.BlockSpec((_B, _D), lambda i: (i, 0)),
            pl.BlockSpec((_D, _D), lambda i: (0, 0)),
            pl.BlockSpec((1, _D), lambda i: (0, 0)),
            pl.BlockSpec((_D, _D), lambda i: (0, 0)),
        ],
        out_specs=pl.BlockSpec((_B, _D), lambda i: (i, 0)),
        out_shape=jax.ShapeDtypeStruct((_N, _D), jnp.float32),
    )(p, degt, h, wl, bl.reshape(1, _D), wr)


def _final_body(p_ref, d_ref, h_ref, wl_ref, bl_ref, wr_ref, b_ref,
                on_ref, og_ref, acc_s, acc_c):
    agg = p_ref[0] + p_ref[1]
    deg = d_ref[0, :, 0] + d_ref[1, :, 0]
    mean = agg * (1.0 / jnp.maximum(deg, 1.0))[:, None]
    xn = (lax.dot_general(mean, wl_ref[...], (((1,), (1,)), ((), ())),
                          preferred_element_type=jnp.float32)
          + lax.dot_general(h_ref[...], wr_ref[...], (((1,), (1,)), ((), ())),
                            preferred_element_type=jnp.float32)
          + bl_ref[...])
    on_ref[...] = xn

    i = pl.program_id(0)
    b = b_ref[0, 0, :]
    oh = (b[:, None] == lax.broadcasted_iota(jnp.int32, (_B, _G), 1)
          ).astype(jnp.float32)
    ps = lax.dot_general(oh, xn, (((0,), (0,)), ((), ())),
                         preferred_element_type=jnp.float32)
    pc = lax.dot_general(oh, jnp.ones_like(xn), (((0,), (0,)), ((), ())),
                         preferred_element_type=jnp.float32)

    @pl.when(i == 0)
    def _():
        acc_s[...] = jnp.zeros_like(acc_s)
        acc_c[...] = jnp.zeros_like(acc_c)

    acc_s[...] += ps
    acc_c[...] += pc

    @pl.when(i == _NSTEPS - 1)
    def _():
        og_ref[...] = acc_s[...] / jnp.maximum(acc_c[...], 1.0)


def _tc_final(p, degt, h, wl, bl, wr, batch3):
    return pl.pallas_call(
        _final_body,
        grid=(_NSTEPS,),
        in_specs=[
            pl.BlockSpec((_NC, _B, _D), lambda i: (0, i, 0)),
            pl.BlockSpec((_NC, _B, _D), lambda i: (0, i, 0)),
            pl.BlockSpec((_B, _D), lambda i: (i, 0)),
            pl.BlockSpec((_D, _D), lambda i: (0, 0)),
            pl.BlockSpec((1, _D), lambda i: (0, 0)),
            pl.BlockSpec((_D, _D), lambda i: (0, 0)),
            pl.BlockSpec((1, 1, _B), lambda i: (i, 0, 0)),
        ],
        out_specs=[
            pl.BlockSpec((_B, _D), lambda i: (i, 0)),
            pl.BlockSpec((_G, _D), lambda i: (0, 0)),
        ],
        out_shape=[
            jax.ShapeDtypeStruct((_N, _D), jnp.float32),
            jax.ShapeDtypeStruct((_G, _D), jnp.float32),
        ],
        scratch_shapes=[
            pltpu.VMEM((_G, _D), jnp.float32),
            pltpu.VMEM((_G, _D), jnp.float32),
        ],
    )(p, degt, h, wl, bl.reshape(1, _D), wr, batch3)


def kernel(x, edge_index, batch,
           Wl1, bl1, Wr1, Wl2, bl2, Wr2, Wl3, bl3, Wr3, Wl4, bl4, Wr4):
    src4 = edge_index[0].reshape(_NC * _NS, _ANR, _ARND, _ACH)
    dst4 = edge_index[1].reshape(_NC * _NS, _ANR, _ARND, _ACH)
    dst2 = edge_index[1].reshape(_NC * _NS, _RPT, _CH)
    batch3 = batch.reshape(_NSTEPS, 1, _B)
    zrows = jnp.zeros((_ZC, _D), jnp.float32)
    orows = jnp.ones((_CH, _D), jnp.float32)

    degt = _sc_deg(dst2, zrows, orows)

    h = x
    for wl, bl, wr in ((Wl1, bl1, Wr1), (Wl2, bl2, Wr2), (Wl3, bl3, Wr3)):
        p = _sc_agg(h, src4, dst4, zrows)
        h = _tc_layer(p, degt, h, wl, bl, wr, relu=True)
    p = _sc_agg(h, src4, dst4, zrows)
    x_nodes, x_graph = _tc_final(p, degt, h, Wl4, bl4, Wr4, batch3)
    return (x_graph, x_nodes)


# speedup vs baseline: 1.1346x; 1.1346x over previous
import functools

import jax
import jax.numpy as jnp
from jax import lax
from jax.experimental import pallas as pl
from jax.experimental.pallas import tpu as pltpu
from jax.experimental.pallas import tpu_sc as plsc

_N, _E, _D, _G = 10000, 320000, 128, 64
_NC, _NS = 2, 16
_CH = 125
_ROWS = _E // _CH
_RPT = _ROWS // (_NC * _NS)
_HR = _RPT // 2
_ZC = 40
_NZC = _N // _ZC
_ACH = 50
_ARND = 50
_ANR = 4
_NBUF = 5
_B = 1000
_NSTEPS = _N // _B

_mesh = plsc.VectorSubcoreMesh(
    core_axis_name="c", subcore_axis_name="s", num_cores=_NC, num_subcores=_NS)


def _agg_body(h_hbm, src_hbm, dst_hbm, zr_hbm, out_hbm,
              sidx, didx, *rest):
    rows = rest[:_NBUF]
    acc = rest[_NBUF]
    gsems = rest[_NBUF + 1:2 * _NBUF + 1]
    ssems = rest[2 * _NBUF + 1:]
    rows0 = rows[0]
    cid = lax.axis_index("c")
    sid = lax.axis_index("s")
    wid = cid * _NS + sid
    pltpu.sync_copy(zr_hbm, rows0.at[pl.ds(0, _ZC)])
    pltpu.sync_copy(src_hbm.at[wid, 0], sidx)
    pltpu.sync_copy(dst_hbm.at[wid, 0], didx)
    for b in range(1, _NBUF - 1):
        pltpu.async_copy(h_hbm.at[sidx.at[b]], rows[b], gsems[b])

    @pl.loop(sid, _NZC, step=_NS)
    def _zero(c):
        pltpu.sync_copy(rows0.at[pl.ds(0, _ZC)], acc.at[pl.ds(c * _ZC, _ZC)])

    pltpu.async_copy(h_hbm.at[sidx.at[0]], rows0, gsems[0])
    plsc.subcore_barrier()

    for r in range(_ANR):
        if r > 0:
            pltpu.sync_copy(src_hbm.at[wid, r], sidx)
            pltpu.sync_copy(dst_hbm.at[wid, r], didx)
            for b in range(_NBUF - 1):
                pltpu.async_copy(h_hbm.at[sidx.at[b]], rows[b], gsems[b])

        @pl.loop(0, _ARND, step=_NBUF)
        def _main(j):
            for b in range(_NBUF):
                c = j + b
                pltpu.make_async_copy(h_hbm.at[sidx.at[c]], rows[b],
                                      gsems[b]).wait()
                nb = (b + _NBUF - 1) % _NBUF

                @pl.when(c + _NBUF - 1 < _ARND)
                def _():
                    if b == 0:
                        @pl.when(j > 0)
                        def _():
                            pltpu.make_async_copy(
                                rows[nb], acc.at[didx.at[c]],
                                ssems[nb]).wait()
                    else:
                        pltpu.make_async_copy(
                            rows[nb], acc.at[didx.at[c]], ssems[nb]).wait()
                    pltpu.async_copy(h_hbm.at[sidx.at[c + _NBUF - 1]],
                                     rows[nb], gsems[nb])

                pltpu.async_copy(rows[b], acc.at[didx.at[c]], ssems[b],
                                 add=True)

        for b in range(_NBUF):
            pltpu.make_async_copy(rows[b], acc.at[didx.at[0]],
                                  ssems[b]).wait()

    plsc.subcore_barrier()

    @pl.loop(sid, _NZC, step=_NS)
    def _wb(c):
        pltpu.sync_copy(acc.at[pl.ds(c * _ZC, _ZC)],
                        out_hbm.at[cid, pl.ds(c * _ZC, _ZC)])


_sc_agg = functools.partial(
    pl.kernel, _agg_body,
    out_type=jax.ShapeDtypeStruct((_NC, _N, _D), jnp.float32),
    mesh=_mesh,
    scratch_types=(
        [pltpu.VMEM((_ARND, _ACH), jnp.int32)] * 2
        + [pltpu.VMEM((_ACH, _D), jnp.float32)] * _NBUF
        + [pltpu.VMEM_SHARED((_N, _D), jnp.float32)]
        + [pltpu.SemaphoreType.DMA] * (2 * _NBUF)
    ),
)()


def _deg_body(dst_hbm, zr_hbm, on_hbm, out_hbm, didx, rows, acc):
    cid = lax.axis_index("c")
    sid = lax.axis_index("s")
    pltpu.sync_copy(zr_hbm, rows.at[pl.ds(0, _ZC)])

    @pl.loop(sid, _NZC, step=_NS)
    def _zero(c):
        pltpu.sync_copy(rows.at[pl.ds(0, _ZC)], acc.at[pl.ds(c * _ZC, _ZC)])

    pltpu.sync_copy(on_hbm, rows)
    plsc.subcore_barrier()

    for half in range(2):
        pltpu.sync_copy(dst_hbm.at[cid * _NS + sid, pl.ds(half * _HR, _HR)],
                        didx)

        @pl.loop(0, _HR)
        def _main(j):
            pltpu.sync_copy(rows, acc.at[didx.at[j]], add=True)

    plsc.subcore_barrier()

    @pl.loop(sid, _NZC, step=_NS)
    def _wb(c):
        pltpu.sync_copy(acc.at[pl.ds(c * _ZC, _ZC)],
                        out_hbm.at[cid, pl.ds(c * _ZC, _ZC)])


_sc_deg = functools.partial(
    pl.kernel, _deg_body,
    out_type=jax.ShapeDtypeStruct((_NC, _N, _D), jnp.float32),
    mesh=_mesh,
    scratch_types=[
        pltpu.VMEM((_HR, _CH), jnp.int32),
        pltpu.VMEM((_CH, _D), jnp.float32),
        pltpu.VMEM_SHARED((_N, _D), jnp.float32),
    ],
)()


def _dense_body(relu, p_ref, d_ref, h_ref, wl_ref, bl_ref, wr_ref, o_ref):
    agg = p_ref[0] + p_ref[1]
    deg = d_ref[0, :, 0] + d_ref[1, :, 0]
    mean = agg * (1.0 / jnp.maximum(deg, 1.0))[:, None]
    out = (lax.dot_general(mean, wl_ref[...], (((1,), (1,)), ((), ())),
                           preferred_element_type=jnp.float32)
           + lax.dot_general(h_ref[...], wr_ref[...], (((1,), (1,)), ((), ())),
                             preferred_element_type=jnp.float32)
           + bl_ref[...])
    if relu:
        out = jnp.maximum(out, 0.0)
    o_ref[...] = out


def _tc_layer(p, degt, h, wl, bl, wr, relu):
    return pl.pallas_call(
        functools.partial(_dense_body, relu),
        grid=(_NSTEPS,),
        in_specs=[
            pl.BlockSpec((_NC, _B, _D), lambda i: (0, i, 0)),
            pl.BlockSpec((_NC, _B, _D), lambda i: (0, i, 0)),
            pl.BlockSpec((_B, _D), lambda i: (i, 0)),
            pl.BlockSpec((_D, _D), lambda i: (0, 0)),
            pl.BlockSpec((1, _D), lambda i: (0, 0)),
            pl.BlockSpec((_D, _D), lambda i: (0, 0)),
        ],
        out_specs=pl.BlockSpec((_B, _D), lambda i: (i, 0)),
        out_shape=jax.ShapeDtypeStruct((_N, _D), jnp.float32),
    )(p, degt, h, wl, bl.reshape(1, _D), wr)


def _final_body(p_ref, d_ref, h_ref, wl_ref, bl_ref, wr_ref, b_ref,
                on_ref, og_ref, acc_s, acc_c):
    agg = p_ref[0] + p_ref[1]
    deg = d_ref[0, :, 0] + d_ref[1, :, 0]
    mean = agg * (1.0 / jnp.maximum(deg, 1.0))[:, None]
    xn = (lax.dot_general(mean, wl_ref[...], (((1,), (1,)), ((), ())),
                          preferred_element_type=jnp.float32)
          + lax.dot_general(h_ref[...], wr_ref[...], (((1,), (1,)), ((), ())),
                            preferred_element_type=jnp.float32)
          + bl_ref[...])
    on_ref[...] = xn

    i = pl.program_id(0)
    b = b_ref[0, 0, :]
    oh = (b[:, None] == lax.broadcasted_iota(jnp.int32, (_B, _G), 1)
          ).astype(jnp.float32)
    ps = lax.dot_general(oh, xn, (((0,), (0,)), ((), ())),
                         preferred_element_type=jnp.float32)
    pc = lax.dot_general(oh, jnp.ones_like(xn), (((0,), (0,)), ((), ())),
                         preferred_element_type=jnp.float32)

    @pl.when(i == 0)
    def _():
        acc_s[...] = jnp.zeros_like(acc_s)
        acc_c[...] = jnp.zeros_like(acc_c)

    acc_s[...] += ps
    acc_c[...] += pc

    @pl.when(i == _NSTEPS - 1)
    def _():
        og_ref[...] = acc_s[...] / jnp.maximum(acc_c[...], 1.0)


def _tc_final(p, degt, h, wl, bl, wr, batch3):
    return pl.pallas_call(
        _final_body,
        grid=(_NSTEPS,),
        in_specs=[
            pl.BlockSpec((_NC, _B, _D), lambda i: (0, i, 0)),
            pl.BlockSpec((_NC, _B, _D), lambda i: (0, i, 0)),
            pl.BlockSpec((_B, _D), lambda i: (i, 0)),
            pl.BlockSpec((_D, _D), lambda i: (0, 0)),
            pl.BlockSpec((1, _D), lambda i: (0, 0)),
            pl.BlockSpec((_D, _D), lambda i: (0, 0)),
            pl.BlockSpec((1, 1, _B), lambda i: (i, 0, 0)),
        ],
        out_specs=[
            pl.BlockSpec((_B, _D), lambda i: (i, 0)),
            pl.BlockSpec((_G, _D), lambda i: (0, 0)),
        ],
        out_shape=[
            jax.ShapeDtypeStruct((_N, _D), jnp.float32),
            jax.ShapeDtypeStruct((_G, _D), jnp.float32),
        ],
        scratch_shapes=[
            pltpu.VMEM((_G, _D), jnp.float32),
            pltpu.VMEM((_G, _D), jnp.float32),
        ],
    )(p, degt, h, wl, bl.reshape(1, _D), wr, batch3)


def kernel(x, edge_index, batch,
           Wl1, bl1, Wr1, Wl2, bl2, Wr2, Wl3, bl3, Wr3, Wl4, bl4, Wr4):
    src4 = edge_index[0].reshape(_NC * _NS, _ANR, _ARND, _ACH)
    dst4 = edge_index[1].reshape(_NC * _NS, _ANR, _ARND, _ACH)
    dst2 = edge_index[1].reshape(_NC * _NS, _RPT, _CH)
    batch3 = batch.reshape(_NSTEPS, 1, _B)
    zrows = jnp.zeros((_ZC, _D), jnp.float32)
    orows = jnp.ones((_CH, _D), jnp.float32)

    degt = _sc_deg(dst2, zrows, orows)

    h = x
    for wl, bl, wr in ((Wl1, bl1, Wr1), (Wl2, bl2, Wr2), (Wl3, bl3, Wr3)):
        p = _sc_agg(h, src4, dst4, zrows)
        h = _tc_layer(p, degt, h, wl, bl, wr, relu=True)
    p = _sc_agg(h, src4, dst4, zrows)
    x_nodes, x_graph = _tc_final(p, degt, h, Wl4, bl4, Wr4, batch3)
    return (x_graph, x_nodes)


# speedup vs baseline: 1.1376x; 1.0027x over previous
import functools

import jax
import jax.numpy as jnp
from jax import lax
from jax.experimental import pallas as pl
from jax.experimental.pallas import tpu as pltpu
from jax.experimental.pallas import tpu_sc as plsc

_N, _E, _D, _G = 10000, 320000, 128, 64
_NC, _NS = 2, 16
_CH = 125
_ROWS = _E // _CH
_RPT = _ROWS // (_NC * _NS)
_HR = _RPT // 2
_ZC = 40
_NZC = _N // _ZC
_ACH = 50
_ARND = 50
_ANR = 4
_NBUF = 5
_B = 1000
_NSTEPS = _N // _B

_mesh = plsc.VectorSubcoreMesh(
    core_axis_name="c", subcore_axis_name="s", num_cores=_NC, num_subcores=_NS)


def _agg_body(h_hbm, src_hbm, dst_hbm, zr_hbm, out_hbm,
              sidx, didx, *rest):
    rows = rest[:_NBUF]
    acc = rest[_NBUF]
    gsems = rest[_NBUF + 1:2 * _NBUF + 1]
    ssems = rest[2 * _NBUF + 1:]
    rows0 = rows[0]
    cid = lax.axis_index("c")
    sid = lax.axis_index("s")
    wid = cid * _NS + sid
    pltpu.sync_copy(zr_hbm, rows0.at[pl.ds(0, _ZC)])
    pltpu.sync_copy(src_hbm.at[wid, 0], sidx)
    pltpu.sync_copy(dst_hbm.at[wid, 0], didx)
    for b in range(1, _NBUF - 1):
        pltpu.async_copy(h_hbm.at[sidx.at[b]], rows[b], gsems[b])

    @pl.loop(sid, _NZC, step=_NS)
    def _zero(c):
        pltpu.sync_copy(rows0.at[pl.ds(0, _ZC)], acc.at[pl.ds(c * _ZC, _ZC)])

    pltpu.async_copy(h_hbm.at[sidx.at[0]], rows0, gsems[0])
    plsc.subcore_barrier()

    for r in range(_ANR):
        if r > 0:
            pltpu.sync_copy(src_hbm.at[wid, r], sidx)
            pltpu.sync_copy(dst_hbm.at[wid, r], didx)
            for b in range(_NBUF - 1):
                pltpu.async_copy(h_hbm.at[sidx.at[b]], rows[b], gsems[b])

        @pl.loop(0, _ARND, step=_NBUF)
        def _main(j):
            for b in range(_NBUF):
                c = j + b
                pltpu.make_async_copy(h_hbm.at[sidx.at[c]], rows[b],
                                      gsems[b]).wait()
                nb = (b + _NBUF - 1) % _NBUF

                @pl.when(c + _NBUF - 1 < _ARND)
                def _():
                    if b == 0:
                        @pl.when(j > 0)
                        def _():
                            pltpu.make_async_copy(
                                rows[nb], acc.at[didx.at[c]],
                                ssems[nb]).wait()
                    else:
                        pltpu.make_async_copy(
                            rows[nb], acc.at[didx.at[c]], ssems[nb]).wait()
                    pltpu.async_copy(h_hbm.at[sidx.at[c + _NBUF - 1]],
                                     rows[nb], gsems[nb])

                pltpu.async_copy(rows[b], acc.at[didx.at[c]], ssems[b],
                                 add=True)

        for b in range(_NBUF):
            pltpu.make_async_copy(rows[b], acc.at[didx.at[0]],
                                  ssems[b]).wait()

    plsc.subcore_barrier()

    @pl.loop(sid, _NZC, step=_NS)
    def _wb(c):
        pltpu.sync_copy(acc.at[pl.ds(c * _ZC, _ZC)],
                        out_hbm.at[cid, pl.ds(c * _ZC, _ZC)])


_sc_agg = functools.partial(
    pl.kernel, _agg_body,
    out_type=jax.ShapeDtypeStruct((_NC, _N, _D), jnp.float32),
    mesh=_mesh,
    scratch_types=(
        [pltpu.VMEM((_ARND, _ACH), jnp.int32)] * 2
        + [pltpu.VMEM((_ACH, _D), jnp.float32)] * _NBUF
        + [pltpu.VMEM_SHARED((_N, _D), jnp.float32)]
        + [pltpu.SemaphoreType.DMA] * (2 * _NBUF)
    ),
)()


def _deg_body(dst_hbm, zr_hbm, on_hbm, out_hbm, didx, rows, acc, ssem):
    cid = lax.axis_index("c")
    sid = lax.axis_index("s")
    pltpu.sync_copy(zr_hbm, rows.at[pl.ds(0, _ZC)])

    @pl.loop(sid, _NZC, step=_NS)
    def _zero(c):
        pltpu.sync_copy(rows.at[pl.ds(0, _ZC)], acc.at[pl.ds(c * _ZC, _ZC)])

    pltpu.sync_copy(on_hbm, rows)
    plsc.subcore_barrier()

    for half in range(2):
        pltpu.sync_copy(dst_hbm.at[cid * _NS + sid, pl.ds(half * _HR, _HR)],
                        didx)

        @pl.loop(0, _HR)
        def _main(j):
            pltpu.async_copy(rows, acc.at[didx.at[j]], ssem, add=True)

        @pl.loop(0, _HR)
        def _drain(j):
            pltpu.make_async_copy(rows, acc.at[didx.at[0]], ssem).wait()

    plsc.subcore_barrier()

    @pl.loop(sid, _NZC, step=_NS)
    def _wb(c):
        pltpu.sync_copy(acc.at[pl.ds(c * _ZC, _ZC)],
                        out_hbm.at[cid, pl.ds(c * _ZC, _ZC)])


_sc_deg = functools.partial(
    pl.kernel, _deg_body,
    out_type=jax.ShapeDtypeStruct((_NC, _N, _D), jnp.float32),
    mesh=_mesh,
    scratch_types=[
        pltpu.VMEM((_HR, _CH), jnp.int32),
        pltpu.VMEM((_CH, _D), jnp.float32),
        pltpu.VMEM_SHARED((_N, _D), jnp.float32),
        pltpu.SemaphoreType.DMA,
    ],
)()


def _dense_body(relu, p_ref, d_ref, h_ref, wl_ref, bl_ref, wr_ref, o_ref):
    agg = p_ref[0] + p_ref[1]
    deg = d_ref[0, :, 0] + d_ref[1, :, 0]
    mean = agg * (1.0 / jnp.maximum(deg, 1.0))[:, None]
    out = (lax.dot_general(mean, wl_ref[...], (((1,), (1,)), ((), ())),
                           preferred_element_type=jnp.float32)
           + lax.dot_general(h_ref[...], wr_ref[...], (((1,), (1,)), ((), ())),
                             preferred_element_type=jnp.float32)
           + bl_ref[...])
    if relu:
        out = jnp.maximum(out, 0.0)
    o_ref[...] = out


def _tc_layer(p, degt, h, wl, bl, wr, relu):
    return pl.pallas_call(
        functools.partial(_dense_body, relu),
        grid=(_NSTEPS,),
        in_specs=[
            pl.BlockSpec((_NC, _B, _D), lambda i: (0, i, 0)),
            pl.BlockSpec((_NC, _B, _D), lambda i: (0, i, 0)),
            pl.BlockSpec((_B, _D), lambda i: (i, 0)),
            pl.BlockSpec((_D, _D), lambda i: (0, 0)),
            pl.BlockSpec((1, _D), lambda i: (0, 0)),
            pl.BlockSpec((_D, _D), lambda i: (0, 0)),
        ],
        out_specs=pl.BlockSpec((_B, _D), lambda i: (i, 0)),
        out_shape=jax.ShapeDtypeStruct((_N, _D), jnp.float32),
    )(p, degt, h, wl, bl.reshape(1, _D), wr)


def _final_body(p_ref, d_ref, h_ref, wl_ref, bl_ref, wr_ref, b_ref,
                on_ref, og_ref, acc_s, acc_c):
    agg = p_ref[0] + p_ref[1]
    deg = d_ref[0, :, 0] + d_ref[1, :, 0]
    mean = agg * (1.0 / jnp.maximum(deg, 1.0))[:, None]
    xn = (lax.dot_general(mean, wl_ref[...], (((1,), (1,)), ((), ())),
                          preferred_element_type=jnp.float32)
          + lax.dot_general(h_ref[...], wr_ref[...], (((1,), (1,)), ((), ())),
                            preferred_element_type=jnp.float32)
          + bl_ref[...])
    on_ref[...] = xn

    i = pl.program_id(0)
    b = b_ref[0, 0, :]
    oh = (b[:, None] == lax.broadcasted_iota(jnp.int32, (_B, _G), 1)
          ).astype(jnp.float32)
    ps = lax.dot_general(oh, xn, (((0,), (0,)), ((), ())),
                         preferred_element_type=jnp.float32)
    pc = lax.dot_general(oh, jnp.ones_like(xn), (((0,), (0,)), ((), ())),
                         preferred_element_type=jnp.float32)

    @pl.when(i == 0)
    def _():
        acc_s[...] = jnp.zeros_like(acc_s)
        acc_c[...] = jnp.zeros_like(acc_c)

    acc_s[...] += ps
    acc_c[...] += pc

    @pl.when(i == _NSTEPS - 1)
    def _():
        og_ref[...] = acc_s[...] / jnp.maximum(acc_c[...], 1.0)


def _tc_final(p, degt, h, wl, bl, wr, batch3):
    return pl.pallas_call(
        _final_body,
        grid=(_NSTEPS,),
        in_specs=[
            pl.BlockSpec((_NC, _B, _D), lambda i: (0, i, 0)),
            pl.BlockSpec((_NC, _B, _D), lambda i: (0, i, 0)),
            pl.BlockSpec((_B, _D), lambda i: (i, 0)),
            pl.BlockSpec((_D, _D), lambda i: (0, 0)),
            pl.BlockSpec((1, _D), lambda i: (0, 0)),
            pl.BlockSpec((_D, _D), lambda i: (0, 0)),
            pl.BlockSpec((1, 1, _B), lambda i: (i, 0, 0)),
        ],
        out_specs=[
            pl.BlockSpec((_B, _D), lambda i: (i, 0)),
            pl.BlockSpec((_G, _D), lambda i: (0, 0)),
        ],
        out_shape=[
            jax.ShapeDtypeStruct((_N, _D), jnp.float32),
            jax.ShapeDtypeStruct((_G, _D), jnp.float32),
        ],
        scratch_shapes=[
            pltpu.VMEM((_G, _D), jnp.float32),
            pltpu.VMEM((_G, _D), jnp.float32),
        ],
    )(p, degt, h, wl, bl.reshape(1, _D), wr, batch3)


def kernel(x, edge_index, batch,
           Wl1, bl1, Wr1, Wl2, bl2, Wr2, Wl3, bl3, Wr3, Wl4, bl4, Wr4):
    src4 = edge_index[0].reshape(_NC * _NS, _ANR, _ARND, _ACH)
    dst4 = edge_index[1].reshape(_NC * _NS, _ANR, _ARND, _ACH)
    dst2 = edge_index[1].reshape(_NC * _NS, _RPT, _CH)
    batch3 = batch.reshape(_NSTEPS, 1, _B)
    zrows = jnp.zeros((_ZC, _D), jnp.float32)
    orows = jnp.ones((_CH, _D), jnp.float32)

    degt = _sc_deg(dst2, zrows, orows)

    h = x
    for wl, bl, wr in ((Wl1, bl1, Wr1), (Wl2, bl2, Wr2), (Wl3, bl3, Wr3)):
        p = _sc_agg(h, src4, dst4, zrows)
        h = _tc_layer(p, degt, h, wl, bl, wr, relu=True)
    p = _sc_agg(h, src4, dst4, zrows)
    x_nodes, x_graph = _tc_final(p, degt, h, Wl4, bl4, Wr4, batch3)
    return (x_graph, x_nodes)
